# baseline (speedup 1.0000x reference)
import jax
import jax.numpy as jnp
from jax import lax
from jax.experimental import pallas as pl
from jax.experimental.pallas import tpu as pltpu

B = 1024
IN_DIM = 768
NUM_TASKS = 8
NUM_EXPERTS = 64
NUM_SHARED = 2
WIDTH = 128
HEAD_DIM = 32
P = 2 * B
TILE = 128
NTILES = P // TILE
NSTEPS = 80


def _dot00(a, b, prefer=jnp.float32):
    return lax.dot_general(a, b, (((0,), (0,)), ((), ())),
                           preferred_element_type=prefer)


def _moe_body(task_ref, gk_ref, rk_any, rb_ref, sk_ref, sb_ref,
              hk_ref, hb_ref, feats_ref, out_ref,
              wv_ref, dma_sem, featsb_ref, xs_ref, hacc_ref, otok_ref,
              offs_ref, eid_ref, tl_ref, tok_ref, ws_ref, tid_ref):
    s = pl.program_id(0)

    @pl.when(s == 0)
    def _prologue():
        pltpu.make_async_copy(rk_any, wv_ref, dma_sem).start()

        task = task_ref[...]
        logits = jnp.dot(task, gk_ref[...],
                         preferred_element_type=jnp.float32)
        iota_e = lax.broadcasted_iota(jnp.int32, (B, NUM_EXPERTS), 1)
        m1 = jnp.max(logits, axis=1, keepdims=True)
        i1 = jnp.min(jnp.where(logits == m1, iota_e, NUM_EXPERTS), axis=1,
                     keepdims=True)
        l2 = jnp.where(iota_e == i1, -jnp.inf, logits)
        m2 = jnp.max(l2, axis=1, keepdims=True)
        i2 = jnp.min(jnp.where(l2 == m2, iota_e, NUM_EXPERTS), axis=1,
                     keepdims=True)
        w1 = 1.0 / (1.0 + jnp.exp(m2 - m1))

        t_iota = lax.broadcasted_iota(jnp.int32, (B, NUM_TASKS), 1)
        tmax = jnp.max(task, axis=1, keepdims=True)
        tid_ref[...] = jnp.min(jnp.where(task == tmax, t_iota, NUM_TASKS),
                               axis=1, keepdims=True)

        o0 = (iota_e == i1).astype(jnp.float32)
        o1 = (iota_e == i2).astype(jnp.float32)
        c0 = jnp.sum(o0, axis=0, keepdims=True)
        c = c0 + jnp.sum(o1, axis=0, keepdims=True)
        r_i = lax.broadcasted_iota(jnp.int32, (B, B), 0)
        c_i = lax.broadcasted_iota(jnp.int32, (B, B), 1)
        lstrict = (c_i < r_i).astype(jnp.float32)
        cc0 = jnp.dot(lstrict, o0, preferred_element_type=jnp.float32)
        cc1 = jnp.dot(lstrict, o1, preferred_element_type=jnp.float32)
        ue_i = lax.broadcasted_iota(jnp.int32, (NUM_EXPERTS, NUM_EXPERTS), 0)
        ue_j = lax.broadcasted_iota(jnp.int32, (NUM_EXPERTS, NUM_EXPERTS), 1)
        ustrict = (ue_i < ue_j).astype(jnp.float32)
        q = jnp.dot(o0 + o1, ustrict, preferred_element_type=jnp.float32)
        off = jnp.sum(q, axis=0, keepdims=True)
        pos0 = jnp.sum(o0 * (off + cc0), axis=1, keepdims=True)
        pos1 = jnp.sum(o1 * (off + c0 + cc1), axis=1, keepdims=True)

        hi_row = off + c
        t0r = jnp.floor(off * (1.0 / TILE))
        t1r = jnp.floor((hi_row - 1.0) * (1.0 / TILE))
        ntr = jnp.where(c > 0, t1r - t0r + 1.0, 0.0)
        gbr = jnp.dot(ntr, ustrict, preferred_element_type=jnp.float32)
        ones11 = jnp.ones((1, 1), jnp.float32)
        gb_c = _dot00(gbr, ones11)
        nt_c = _dot00(ntr, ones11)
        t0_c = _dot00(t0r, ones11)
        e_colf = lax.broadcasted_iota(jnp.int32, (NUM_EXPERTS, 1), 0).astype(
            jnp.float32)
        s_row = lax.broadcasted_iota(jnp.int32, (1, 128), 1)
        g_row = jnp.where(s_row < NSTEPS,
                          (s_row % 20) * 4 + s_row // 20,
                          127).astype(jnp.float32)
        inr = ((g_row >= gb_c) & (g_row < gb_c + nt_c)).astype(jnp.float32)
        cover = jnp.sum(inr, axis=0, keepdims=True)
        eid = jnp.sum(inr * e_colf, axis=0, keepdims=True) + NUM_EXPERTS * (
            1.0 - cover)
        tl = jnp.sum(inr * (g_row - gb_c + t0_c), axis=0, keepdims=True) + (
            NTILES - 1.0) * (1.0 - cover)
        eid_ref[...] = eid.astype(jnp.int32)
        tl_ref[...] = tl.astype(jnp.int32)
        lane128 = lax.broadcasted_iota(jnp.int32, (1, 128), 1)
        offp = jnp.concatenate(
            [off, jnp.full((1, 128 - NUM_EXPERTS), float(P), jnp.float32)],
            axis=1)
        offs_ref[...] = jnp.where(lane128 < NUM_EXPERTS, offp,
                                  float(P)).astype(jnp.int32)

        tok_i = lax.broadcasted_iota(jnp.int32, (B, 1), 0)
        tok_hi = (tok_i // 64).astype(jnp.float32)
        tok_lo = (tok_i % 64).astype(jnp.float32)
        w2 = 1.0 - w1

        def _payload(w):
            wa = w.astype(jnp.bfloat16).astype(jnp.float32)
            return jnp.concatenate([tok_hi, tok_lo, wa, w - wa], axis=1)

        vals0 = _payload(w1)
        vals1 = _payload(w2)
        lane = lax.broadcasted_iota(jnp.int32, (1, TILE), 1).astype(
            jnp.float32)
        featsb_ref[...] = feats_ref[...].astype(jnp.bfloat16)
        t_row = lax.broadcasted_iota(jnp.int32, (1, B), 1)
        for j in range(NTILES):
            p_row = lane + (j * TILE)
            m0 = (pos0 == p_row).astype(jnp.float32)
            m1h = (pos1 == p_row).astype(jnp.float32)
            st = _dot00(m0, vals0) + _dot00(m1h, vals1)
            tok_t = (st[:, 0:1] * 64.0 + st[:, 1:2]).astype(jnp.int32)
            tok_ref[pl.ds(j * TILE, TILE), :] = tok_t
            ws_ref[pl.ds(j * TILE, TILE), :] = st[:, 2:3] + st[:, 3:4]
            mg = (tok_t == t_row).astype(jnp.bfloat16)
            xs_ref[pl.ds(j * TILE, TILE), :] = jnp.dot(
                mg, featsb_ref[...],
                preferred_element_type=jnp.float32)

        hacc_ref[...] = jnp.zeros((P, WIDTH), jnp.float32)
        sh = jnp.zeros((B, WIDTH), jnp.float32)
        for j in range(NUM_SHARED):
            h = jnp.dot(featsb_ref[...], sk_ref[j].astype(jnp.bfloat16),
                        preferred_element_type=jnp.float32)
            sh = sh + jax.nn.relu(h + sb_ref[j][None, :])
        otok_ref[...] = sh * (1.0 / NUM_SHARED)

    @pl.when(s == 1)
    def _wait_weights():
        pltpu.make_async_copy(rk_any, wv_ref, dma_sem).wait()

    @pl.when((s >= 1) & (s <= NSTEPS))
    def _work_step():
        g = s - 1
        lane = lax.broadcasted_iota(jnp.int32, (1, 128), 1)
        e = jnp.sum(jnp.where(lane == g, eid_ref[...], 0))
        tlv = jnp.sum(jnp.where(lane == g, tl_ref[...], 0))
        lo = jnp.sum(jnp.where(lane == e, offs_ref[...], 0))
        hi = jnp.sum(jnp.where(lane == e + 1, offs_ref[...], 0))
        emin = jnp.minimum(e, NUM_EXPERTS - 1)
        base = tlv * TILE
        w_b = wv_ref[pl.ds(emin, 1)][0]
        b_e = rb_ref[pl.ds(emin, 1), :]
        row_iota = lax.broadcasted_iota(jnp.int32, (TILE, 1), 0)

        xt = xs_ref[pl.ds(base, TILE), :]
        h = jnp.dot(xt, w_b, preferred_element_type=jnp.float32)
        h = jax.nn.relu(h + b_e)
        p_glob = base + row_iota
        mask = (p_glob >= lo) & (p_glob < hi)
        wrow = ws_ref[pl.ds(base, TILE), :]
        contrib = jnp.where(mask, wrow * h, 0.0)
        hacc_ref[pl.ds(base, TILE), :] += contrib

    @pl.when(s == NSTEPS + 1)
    def _epilogue():
        t_row = lax.broadcasted_iota(jnp.int32, (1, B), 1)
        acc = otok_ref[...]
        for j in range(NTILES):
            tok_t = tok_ref[pl.ds(j * TILE, TILE), :]
            mc = (tok_t == t_row).astype(jnp.float32)
            acc = acc + _dot00(mc, hacc_ref[pl.ds(j * TILE, TILE), :])
        f = jnp.tanh(acc)
        heads = jnp.dot(f, hk_ref[...], preferred_element_type=jnp.float32)
        heads = heads + hb_ref[...]
        cols = lax.broadcasted_iota(jnp.int32, (B, NUM_TASKS * HEAD_DIM), 1)
        sel = jnp.where(cols // HEAD_DIM == tid_ref[...], heads, 0.0)
        fold = (lax.broadcasted_iota(jnp.int32, (NUM_TASKS * HEAD_DIM, HEAD_DIM), 0) % HEAD_DIM
                == lax.broadcasted_iota(jnp.int32, (NUM_TASKS * HEAD_DIM, HEAD_DIM), 1)
                ).astype(jnp.float32)
        out_ref[...] = jnp.dot(sel, fold, preferred_element_type=jnp.float32)


@jax.jit
def kernel(x, gating_kernel, routed_kernel_0, routed_bias_0,
           shared_kernel_0, shared_bias_0, head_kernel, head_bias):
    feats = x[:, :IN_DIM]
    task = x[:, IN_DIM:]
    hk2 = head_kernel.transpose(1, 0, 2).reshape(WIDTH, NUM_TASKS * HEAD_DIM)
    hb2 = head_bias.reshape(1, NUM_TASKS * HEAD_DIM)

    full = lambda shape: pl.BlockSpec(shape, lambda i: (0,) * len(shape))
    return pl.pallas_call(
        _moe_body,
        grid=(NSTEPS + 2,),
        in_specs=[
            full((B, NUM_TASKS)),
            full((NUM_TASKS, NUM_EXPERTS)),
            pl.BlockSpec(memory_space=pl.ANY),
            full((NUM_EXPERTS, WIDTH)),
            full((NUM_SHARED, IN_DIM, WIDTH)),
            full((NUM_SHARED, WIDTH)),
            full((WIDTH, NUM_TASKS * HEAD_DIM)),
            full((1, NUM_TASKS * HEAD_DIM)),
            full((B, IN_DIM)),
        ],
        out_specs=full((B, HEAD_DIM)),
        out_shape=jax.ShapeDtypeStruct((B, HEAD_DIM), jnp.float32),
        scratch_shapes=[
            pltpu.VMEM((NUM_EXPERTS, IN_DIM, WIDTH), jnp.float32),
            pltpu.SemaphoreType.DMA,
            pltpu.VMEM((B, IN_DIM), jnp.bfloat16),
            pltpu.VMEM((P, IN_DIM), jnp.float32),
            pltpu.VMEM((P, WIDTH), jnp.float32),
            pltpu.VMEM((B, WIDTH), jnp.float32),
            pltpu.VMEM((1, 128), jnp.int32),
            pltpu.VMEM((1, 128), jnp.int32),
            pltpu.VMEM((1, 128), jnp.int32),
            pltpu.VMEM((P, 1), jnp.int32),
            pltpu.VMEM((P, 1), jnp.float32),
            pltpu.VMEM((B, 1), jnp.int32),
        ],
        compiler_params=pltpu.CompilerParams(
            dimension_semantics=("arbitrary",)),
    )(task, gating_kernel, routed_kernel_0, routed_bias_0,
      shared_kernel_0, shared_bias_0, hk2, hb2, feats)


# speedup vs baseline: 1.0506x; 1.0506x over previous
import jax
import jax.numpy as jnp
from jax import lax
from jax.experimental import pallas as pl
from jax.experimental.pallas import tpu as pltpu

B = 1024
IN_DIM = 768
NUM_TASKS = 8
NUM_EXPERTS = 64
NUM_SHARED = 2
WIDTH = 128
HEAD_DIM = 32
P = 2 * B
TILE = 128
NTILES = P // TILE
NSTEPS = 80


def _dot00(a, b, prefer=jnp.float32):
    return lax.dot_general(a, b, (((0,), (0,)), ((), ())),
                           preferred_element_type=prefer)


def _moe_body(task_ref, gk_ref, rk_any, rb_ref, sk_ref, sb_ref,
              hk_ref, hb_ref, feats_ref, out_ref,
              wv_ref, dma_sem, featsb_ref, xs_ref, hacc_ref, otok_ref,
              offs_ref, eid_ref, tl_ref, tok_ref, ws_ref, tid_ref):
    s = pl.program_id(0)

    @pl.when(s == 0)
    def _prologue():
        pltpu.make_async_copy(rk_any, wv_ref, dma_sem).start()

        task = task_ref[...]
        logits = jnp.dot(task, gk_ref[...],
                         preferred_element_type=jnp.float32)
        iota_e = lax.broadcasted_iota(jnp.int32, (B, NUM_EXPERTS), 1)
        m1 = jnp.max(logits, axis=1, keepdims=True)
        i1 = jnp.min(jnp.where(logits == m1, iota_e, NUM_EXPERTS), axis=1,
                     keepdims=True)
        l2 = jnp.where(iota_e == i1, -jnp.inf, logits)
        m2 = jnp.max(l2, axis=1, keepdims=True)
        i2 = jnp.min(jnp.where(l2 == m2, iota_e, NUM_EXPERTS), axis=1,
                     keepdims=True)
        w1 = 1.0 / (1.0 + jnp.exp(m2 - m1))

        t_iota = lax.broadcasted_iota(jnp.int32, (B, NUM_TASKS), 1)
        tmax = jnp.max(task, axis=1, keepdims=True)
        tid_ref[...] = jnp.min(jnp.where(task == tmax, t_iota, NUM_TASKS),
                               axis=1, keepdims=True)

        o0 = (iota_e == i1).astype(jnp.float32)
        o1 = (iota_e == i2).astype(jnp.float32)
        c0 = jnp.sum(o0, axis=0, keepdims=True)
        c = c0 + jnp.sum(o1, axis=0, keepdims=True)
        r_i = lax.broadcasted_iota(jnp.int32, (B, B), 0)
        c_i = lax.broadcasted_iota(jnp.int32, (B, B), 1)
        lstrict = (c_i < r_i).astype(jnp.float32)
        cc0 = jnp.dot(lstrict, o0, preferred_element_type=jnp.float32)
        cc1 = jnp.dot(lstrict, o1, preferred_element_type=jnp.float32)
        ue_i = lax.broadcasted_iota(jnp.int32, (NUM_EXPERTS, NUM_EXPERTS), 0)
        ue_j = lax.broadcasted_iota(jnp.int32, (NUM_EXPERTS, NUM_EXPERTS), 1)
        ustrict = (ue_i < ue_j).astype(jnp.float32)
        q = jnp.dot(o0 + o1, ustrict, preferred_element_type=jnp.float32)
        off = jnp.sum(q, axis=0, keepdims=True)
        pos0 = jnp.sum(o0 * (off + cc0), axis=1, keepdims=True)
        pos1 = jnp.sum(o1 * (off + c0 + cc1), axis=1, keepdims=True)

        hi_row = off + c
        t0r = jnp.floor(off * (1.0 / TILE))
        t1r = jnp.floor((hi_row - 1.0) * (1.0 / TILE))
        ntr = jnp.where(c > 0, t1r - t0r + 1.0, 0.0)
        gbr = jnp.dot(ntr, ustrict, preferred_element_type=jnp.float32)
        ones11 = jnp.ones((1, 1), jnp.float32)
        gb_c = _dot00(gbr, ones11)
        nt_c = _dot00(ntr, ones11)
        t0_c = _dot00(t0r, ones11)
        e_colf = lax.broadcasted_iota(jnp.int32, (NUM_EXPERTS, 1), 0).astype(
            jnp.float32)
        s_row = lax.broadcasted_iota(jnp.int32, (1, 128), 1)
        g_row = jnp.where(s_row < NSTEPS,
                          (s_row % 20) * 4 + s_row // 20,
                          127).astype(jnp.float32)
        inr = ((g_row >= gb_c) & (g_row < gb_c + nt_c)).astype(jnp.float32)
        cover = jnp.sum(inr, axis=0, keepdims=True)
        eid = jnp.sum(inr * e_colf, axis=0, keepdims=True) + NUM_EXPERTS * (
            1.0 - cover)
        tl = jnp.sum(inr * (g_row - gb_c + t0_c), axis=0, keepdims=True) + (
            NTILES - 1.0) * (1.0 - cover)
        eid_ref[...] = eid.astype(jnp.int32)
        tl_ref[...] = tl.astype(jnp.int32)
        lane128 = lax.broadcasted_iota(jnp.int32, (1, 128), 1)
        offp = jnp.concatenate(
            [off, jnp.full((1, 128 - NUM_EXPERTS), float(P), jnp.float32)],
            axis=1)
        offs_ref[...] = jnp.where(lane128 < NUM_EXPERTS, offp,
                                  float(P)).astype(jnp.int32)

        tok_i = lax.broadcasted_iota(jnp.int32, (B, 1), 0)
        tok_hi = (tok_i // 64).astype(jnp.float32)
        tok_lo = (tok_i % 64).astype(jnp.float32)
        w2 = 1.0 - w1

        def _payload(w):
            wa = w.astype(jnp.bfloat16).astype(jnp.float32)
            return jnp.concatenate([tok_hi, tok_lo, wa, w - wa], axis=1)

        vals0 = _payload(w1)
        vals1 = _payload(w2)
        lane = lax.broadcasted_iota(jnp.int32, (1, TILE), 1).astype(
            jnp.float32)
        featsb_ref[...] = feats_ref[...].astype(jnp.bfloat16)
        t_row = lax.broadcasted_iota(jnp.int32, (1, B), 1)
        for j in range(NTILES):
            p_row = lane + (j * TILE)
            m0 = (pos0 == p_row).astype(jnp.float32)
            m1h = (pos1 == p_row).astype(jnp.float32)
            st = _dot00(m0, vals0) + _dot00(m1h, vals1)
            tok_t = (st[:, 0:1] * 64.0 + st[:, 1:2]).astype(jnp.int32)
            tok_ref[pl.ds(j * TILE, TILE), :] = tok_t
            ws_ref[pl.ds(j * TILE, TILE), :] = st[:, 2:3] + st[:, 3:4]
            mg = (tok_t == t_row).astype(jnp.bfloat16)
            xs_ref[pl.ds(j * TILE, TILE), :] = jnp.dot(
                mg, featsb_ref[...],
                preferred_element_type=jnp.float32)

        hacc_ref[...] = jnp.zeros((P, WIDTH), jnp.float32)
        sh = jnp.zeros((B, WIDTH), jnp.float32)
        for j in range(NUM_SHARED):
            h = jnp.dot(featsb_ref[...], sk_ref[j].astype(jnp.bfloat16),
                        preferred_element_type=jnp.float32)
            sh = sh + jax.nn.relu(h + sb_ref[j][None, :])
        otok_ref[...] = sh * (1.0 / NUM_SHARED)

    @pl.when(s == 1)
    def _work_steps():
        pltpu.make_async_copy(rk_any, wv_ref, dma_sem).wait()
        lane = lax.broadcasted_iota(jnp.int32, (1, 128), 1)
        row_iota = lax.broadcasted_iota(jnp.int32, (TILE, 1), 0)

        def step(g, carry):
            e = jnp.sum(jnp.where(lane == g, eid_ref[...], 0))
            tlv = jnp.sum(jnp.where(lane == g, tl_ref[...], 0))
            lo = jnp.sum(jnp.where(lane == e, offs_ref[...], 0))
            hi = jnp.sum(jnp.where(lane == e + 1, offs_ref[...], 0))
            emin = jnp.minimum(e, NUM_EXPERTS - 1)
            base = tlv * TILE
            w_b = wv_ref[pl.ds(emin, 1)][0]
            b_e = rb_ref[pl.ds(emin, 1), :]
            xt = xs_ref[pl.ds(base, TILE), :]
            h = jnp.dot(xt, w_b, preferred_element_type=jnp.float32)
            h = jax.nn.relu(h + b_e)
            p_glob = base + row_iota
            mask = (p_glob >= lo) & (p_glob < hi)
            wrow = ws_ref[pl.ds(base, TILE), :]
            contrib = jnp.where(mask, wrow * h, 0.0)
            hacc_ref[pl.ds(base, TILE), :] += contrib
            return carry

        lax.fori_loop(0, NSTEPS, step, 0)

    @pl.when(s == 2)
    def _epilogue():
        t_row = lax.broadcasted_iota(jnp.int32, (1, B), 1)
        acc = otok_ref[...]
        for j in range(NTILES):
            tok_t = tok_ref[pl.ds(j * TILE, TILE), :]
            mc = (tok_t == t_row).astype(jnp.float32)
            acc = acc + _dot00(mc, hacc_ref[pl.ds(j * TILE, TILE), :])
        f = jnp.tanh(acc)
        heads = jnp.dot(f, hk_ref[...], preferred_element_type=jnp.float32)
        heads = heads + hb_ref[...]
        cols = lax.broadcasted_iota(jnp.int32, (B, NUM_TASKS * HEAD_DIM), 1)
        sel = jnp.where(cols // HEAD_DIM == tid_ref[...], heads, 0.0)
        fold = (lax.broadcasted_iota(jnp.int32, (NUM_TASKS * HEAD_DIM, HEAD_DIM), 0) % HEAD_DIM
                == lax.broadcasted_iota(jnp.int32, (NUM_TASKS * HEAD_DIM, HEAD_DIM), 1)
                ).astype(jnp.float32)
        out_ref[...] = jnp.dot(sel, fold, preferred_element_type=jnp.float32)


@jax.jit
def kernel(x, gating_kernel, routed_kernel_0, routed_bias_0,
           shared_kernel_0, shared_bias_0, head_kernel, head_bias):
    feats = x[:, :IN_DIM]
    task = x[:, IN_DIM:]
    hk2 = head_kernel.transpose(1, 0, 2).reshape(WIDTH, NUM_TASKS * HEAD_DIM)
    hb2 = head_bias.reshape(1, NUM_TASKS * HEAD_DIM)

    full = lambda shape: pl.BlockSpec(shape, lambda i: (0,) * len(shape))
    return pl.pallas_call(
        _moe_body,
        grid=(3,),
        in_specs=[
            full((B, NUM_TASKS)),
            full((NUM_TASKS, NUM_EXPERTS)),
            pl.BlockSpec(memory_space=pl.ANY),
            full((NUM_EXPERTS, WIDTH)),
            full((NUM_SHARED, IN_DIM, WIDTH)),
            full((NUM_SHARED, WIDTH)),
            full((WIDTH, NUM_TASKS * HEAD_DIM)),
            full((1, NUM_TASKS * HEAD_DIM)),
            full((B, IN_DIM)),
        ],
        out_specs=full((B, HEAD_DIM)),
        out_shape=jax.ShapeDtypeStruct((B, HEAD_DIM), jnp.float32),
        scratch_shapes=[
            pltpu.VMEM((NUM_EXPERTS, IN_DIM, WIDTH), jnp.float32),
            pltpu.SemaphoreType.DMA,
            pltpu.VMEM((B, IN_DIM), jnp.bfloat16),
            pltpu.VMEM((P, IN_DIM), jnp.float32),
            pltpu.VMEM((P, WIDTH), jnp.float32),
            pltpu.VMEM((B, WIDTH), jnp.float32),
            pltpu.VMEM((1, 128), jnp.int32),
            pltpu.VMEM((1, 128), jnp.int32),
            pltpu.VMEM((1, 128), jnp.int32),
            pltpu.VMEM((P, 1), jnp.int32),
            pltpu.VMEM((P, 1), jnp.float32),
            pltpu.VMEM((B, 1), jnp.int32),
        ],
        compiler_params=pltpu.CompilerParams(
            dimension_semantics=("arbitrary",)),
    )(task, gating_kernel, routed_kernel_0, routed_bias_0,
      shared_kernel_0, shared_bias_0, hk2, hb2, feats)
